# baseline (speedup 1.0000x reference)
import functools

import jax
import jax.numpy as jnp
from jax import lax
from jax.experimental import pallas as pl
from jax.experimental.pallas import tpu as pltpu
from jax.experimental.pallas import tpu_sc as plsc

_D = 64
_CHUNK = 128


@functools.partial(jax.jit, static_argnames=("num_rows",))
def _sc_gather(weight, flat_idx, num_rows):
    info = plsc.get_sparse_core_info()
    nw = info.num_cores * info.num_subcores
    rows_per_w = num_rows // nw
    chunks_per_w = rows_per_w // _CHUNK
    mesh = plsc.VectorSubcoreMesh(core_axis_name="c", subcore_axis_name="s")

    @functools.partial(
        pl.kernel,
        mesh=mesh,
        out_type=jax.ShapeDtypeStruct((num_rows, _D), jnp.float32),
        compiler_params=pltpu.CompilerParams(use_tc_tiling_on_sc=False),
        scratch_types=[
            pltpu.VMEM((_CHUNK,), jnp.int32),
            pltpu.VMEM((_CHUNK, _D), jnp.float32),
            pltpu.SemaphoreType.DMA,
        ],
    )
    def k(table_hbm, idx_hbm, out_hbm, idx_v, rows_v, sem):
        wid = lax.axis_index("s") * info.num_cores + lax.axis_index("c")
        base = wid * rows_per_w

        def body(i, carry):
            off = pl.multiple_of(base + i * _CHUNK, _CHUNK)
            pltpu.sync_copy(idx_hbm.at[pl.ds(off, _CHUNK)], idx_v)
            pltpu.async_copy(table_hbm.at[idx_v], rows_v, sem).wait()
            pltpu.sync_copy(rows_v, out_hbm.at[pl.ds(off, _CHUNK)])
            return carry

        lax.fori_loop(0, chunks_per_w, body, 0)

    return k(weight, flat_idx)


def kernel(token_ids, weight):
    b, s = token_ids.shape
    num_rows = b * s
    flat_idx = token_ids.reshape(num_rows).astype(jnp.int32)
    out = _sc_gather(weight, flat_idx, num_rows)
    return out.reshape(b, s, _D)


# speedup vs baseline: 1.1949x; 1.1949x over previous
import functools

import jax
import jax.numpy as jnp
from jax import lax
from jax.experimental import pallas as pl
from jax.experimental.pallas import tpu as pltpu
from jax.experimental.pallas import tpu_sc as plsc

_D = 64
_G = 128
_K = 5
_MC = _K * _G
_NBUF = 2


@functools.partial(jax.jit, static_argnames=("num_rows",))
def _sc_gather(weight, flat_idx, num_rows):
    info = plsc.get_sparse_core_info()
    nw = info.num_cores * info.num_subcores
    rows_per_w = num_rows // nw
    n_macro = rows_per_w // _MC
    n_groups = rows_per_w // _G
    mesh = plsc.VectorSubcoreMesh(core_axis_name="c", subcore_axis_name="s")

    @functools.partial(
        pl.kernel,
        mesh=mesh,
        out_type=jax.ShapeDtypeStruct((num_rows, _D), jnp.float32),
        compiler_params=pltpu.CompilerParams(use_tc_tiling_on_sc=False),
        scratch_types=[
            pltpu.VMEM((n_groups, _G), jnp.int32),
            pltpu.VMEM((_NBUF, _MC, _D), jnp.float32),
            pltpu.SemaphoreType.DMA((_NBUF,)),
            pltpu.SemaphoreType.DMA((_NBUF,)),
        ],
    )
    def k(table_hbm, idx_hbm, out_hbm, idx_v, rows_v, gsem, ssem):
        wid = lax.axis_index("s") * info.num_cores + lax.axis_index("c")
        base = wid * rows_per_w
        pltpu.sync_copy(
            idx_hbm.at[pl.ds(pl.multiple_of(wid * n_groups, 8), n_groups)], idx_v
        )

        def fire(m, b):
            for j in range(_K):
                pltpu.async_copy(
                    table_hbm.at[idx_v.at[m * _K + j]],
                    rows_v.at[b, pl.ds(j * _G, _G)],
                    gsem.at[b],
                )

        def drain_and_store(m, b):
            pltpu.make_async_copy(
                table_hbm.at[pl.ds(0, _MC)], rows_v.at[b], gsem.at[b]
            ).wait()
            off = pl.multiple_of(base + m * _MC, _MC)
            pltpu.async_copy(rows_v.at[b], out_hbm.at[pl.ds(off, _MC)], ssem.at[b])

        def wait_store(b):
            pltpu.make_async_copy(
                table_hbm.at[pl.ds(0, _MC)], rows_v.at[b], ssem.at[b]
            ).wait()

        fire(0, 0)

        @pl.loop(0, n_macro, step=_NBUF)
        def _pair(i):
            @pl.when(i + 1 < n_macro)
            def _():
                pl.when(i + 1 >= _NBUF + 1)(lambda: wait_store(1))
                fire(i + 1, 1)

            drain_and_store(i, 0)

            @pl.when(i + _NBUF < n_macro)
            def _():
                wait_store(0)
                fire(i + _NBUF, 0)

            pl.when(i + 1 < n_macro)(lambda: drain_and_store(i + 1, 1))

        wait_store(0)
        pl.when(n_macro > 1)(lambda: wait_store(1))

    return k(weight, flat_idx)


def kernel(token_ids, weight):
    b, s = token_ids.shape
    num_rows = b * s
    flat_idx = token_ids.astype(jnp.int32).reshape(num_rows // _G, _G)
    out = _sc_gather(weight, flat_idx, num_rows)
    return out.reshape(b, s, _D)
